# baseline (speedup 1.0000x reference)
import functools

import jax
import jax.numpy as jnp
from jax import lax
from jax.experimental import pallas as pl
from jax.experimental.pallas import tpu as pltpu
from jax.experimental.pallas import tpu_sc as plsc

_NUM_CORES = 2
_NUM_SUBCORES = 16
_NUM_WORKERS = _NUM_CORES * _NUM_SUBCORES
_LANES = 16


def _top2(logits, e):
    tt = logits.shape[0]
    iota = jax.lax.broadcasted_iota(jnp.int32, (tt, e), 1)
    m1 = jnp.max(logits, axis=1, keepdims=True)
    idx1 = jnp.min(jnp.where(logits == m1, iota, e), axis=1, keepdims=True)
    masked = jnp.where(iota == idx1, -jnp.inf, logits)
    m2 = jnp.max(masked, axis=1, keepdims=True)
    idx2 = jnp.min(jnp.where(masked == m2, iota, e), axis=1, keepdims=True)
    return iota, m1, idx1, m2, idx2


def _logits_body(x_ref, w_ref, out_ref, map_ref):
    logits = jax.lax.dot_general(
        x_ref[...], w_ref[...], (((1,), (1,)), ((), ())),
        preferred_element_type=jnp.float32,
    )
    out_ref[...] = logits
    iota, _, idx1, _, idx2 = _top2(logits, logits.shape[1])
    map_ref[...] = (iota == idx1) | (iota == idx2)


@functools.partial(jax.jit, static_argnames=("tt", "chunk", "cidx"))
def _logits_tc(x, w, tt, chunk, cidx):
    _, d = x.shape
    e = w.shape[0]
    off = cidx * (chunk // tt)
    return pl.pallas_call(
        _logits_body,
        grid=(chunk // tt,),
        in_specs=[
            pl.BlockSpec((tt, d), lambda i: (i + off, 0)),
            pl.BlockSpec((e, d), lambda i: (0, 0)),
        ],
        out_specs=[
            pl.BlockSpec((tt, e), lambda i: (i, 0)),
            pl.BlockSpec((tt, e), lambda i: (i, 0)),
        ],
        out_shape=[
            jax.ShapeDtypeStruct((chunk, e), jnp.float32),
            jax.ShapeDtypeStruct((chunk, e), jnp.bool_),
        ],
        compiler_params=pltpu.CompilerParams(
            dimension_semantics=("arbitrary",),
        ),
    )(x, w)


def _make_sc_router(tokens, e):
    tpw = tokens // _NUM_WORKERS
    mesh = plsc.VectorSubcoreMesh(core_axis_name="c", subcore_axis_name="s")

    @functools.partial(
        pl.kernel,
        mesh=mesh,
        out_type=jax.ShapeDtypeStruct((tokens, e), jnp.float32),
        scratch_types=[
            pltpu.VMEM((tpw, e), jnp.float32),
            pltpu.VMEM((tpw, e), jnp.float32),
            pltpu.SemaphoreType.DMA,
        ],
        compiler_params=pltpu.CompilerParams(needs_layout_passes=False),
    )
    def _sc_route(logits_hbm, probs_hbm, lbuf, pbuf, sem):
        wid = lax.axis_index("s") * _NUM_CORES + lax.axis_index("c")
        base = wid * tpw
        pltpu.async_copy(logits_hbm.at[pl.ds(base, tpw)], lbuf, sem).wait()

        iota = lax.broadcasted_iota(jnp.int32, (_LANES,), 0)
        neg_inf = jnp.full((_LANES,), -jnp.inf, jnp.float32)
        zero = jnp.zeros((_LANES,), jnp.float32)
        big = jnp.full((_LANES,), e, jnp.int32)

        @pl.loop(0, tpw // _LANES)
        def _(g):
            rows = g * _LANES + iota
            cols = [jnp.full((_LANES,), ee, jnp.int32) for ee in range(e)]
            v = [plsc.load_gather(lbuf, [rows, cols[ee]]) for ee in range(e)]
            m1 = v[0]
            for ee in range(1, e):
                m1 = jnp.maximum(m1, v[ee])
            idx1 = big
            for ee in range(e):
                idx1 = jnp.minimum(
                    idx1, jnp.where(v[ee] == m1, cols[ee], big)
                )
            sel1 = [idx1 == ee for ee in range(e)]
            v2 = [jnp.where(sel1[ee], neg_inf, v[ee]) for ee in range(e)]
            m2 = v2[0]
            for ee in range(1, e):
                m2 = jnp.maximum(m2, v2[ee])
            idx2 = big
            for ee in range(e):
                idx2 = jnp.minimum(
                    idx2, jnp.where(v2[ee] == m2, cols[ee], big)
                )
            tv = jnp.exp(m2 - m1)
            denom = 1.0 + tv
            p1 = 1.0 / denom
            p2 = tv / denom
            for ee in range(e):
                pe = jnp.where(sel1[ee], p1, jnp.where(idx2 == ee, p2, zero))
                plsc.store_scatter(pbuf, [rows, cols[ee]], pe)

        pltpu.async_copy(pbuf, probs_hbm.at[pl.ds(base, tpw)], sem).wait()

    return _sc_route


@jax.jit
def _route_hybrid(x, w):
    tokens, _ = x.shape
    e = w.shape[0]
    nchunks = 2
    chunk = tokens // nchunks
    sc_router = _make_sc_router(chunk, e)
    probs_parts, map_parts = [], []
    for c in range(nchunks):
        logits_c, map_c = _logits_tc(x, w, tt=1024, chunk=chunk, cidx=c)
        probs_parts.append(sc_router(logits_c))
        map_parts.append(map_c)
    return (
        jnp.concatenate(probs_parts, axis=0),
        jnp.concatenate(map_parts, axis=0),
    )


def kernel(hidden_states, router_weight):
    s, b, d = hidden_states.shape
    x = hidden_states.reshape(s * b, d).astype(jnp.float32)
    return _route_hybrid(x, router_weight.astype(jnp.float32))


# speedup vs baseline: 3.2472x; 3.2472x over previous
import functools

import jax
import jax.numpy as jnp
from jax import lax
from jax.experimental import pallas as pl
from jax.experimental.pallas import tpu as pltpu
from jax.experimental.pallas import tpu_sc as plsc

_NUM_CORES = 2
_NUM_SUBCORES = 16
_NUM_WORKERS = _NUM_CORES * _NUM_SUBCORES
_LANES = 16


def _top2(logits, e):
    tt = logits.shape[0]
    iota = jax.lax.broadcasted_iota(jnp.int32, (tt, e), 1)
    m1 = jnp.max(logits, axis=1, keepdims=True)
    idx1 = jnp.min(jnp.where(logits == m1, iota, e), axis=1, keepdims=True)
    masked = jnp.where(iota == idx1, -jnp.inf, logits)
    m2 = jnp.max(masked, axis=1, keepdims=True)
    idx2 = jnp.min(jnp.where(masked == m2, iota, e), axis=1, keepdims=True)
    return iota, m1, idx1, m2, idx2


def _fused_body(x_ref, w_ref, probs_ref, map_ref):
    ts, b, d = x_ref.shape
    e = w_ref.shape[0]
    x = x_ref[...].reshape(ts * b, d)
    logits = jax.lax.dot_general(
        x, w_ref[...], (((1,), (1,)), ((), ())),
        preferred_element_type=jnp.float32,
    )
    iota, m1, idx1, m2, idx2 = _top2(logits, e)
    t = jnp.exp(m2 - m1)
    denom = 1.0 + t
    p1 = 1.0 / denom
    p2 = t / denom
    probs = jnp.where(iota == idx1, p1, jnp.where(iota == idx2, p2, 0.0))
    rmap = (iota == idx1) | (iota == idx2)
    probs_ref[...] = probs.reshape(ts, b, e)
    map_ref[...] = rmap.reshape(ts, b, e)


@functools.partial(jax.jit, static_argnames=("ts",))
def _route_fused3d(h, w, ts):
    s, b, d = h.shape
    e = w.shape[0]
    probs3, map3 = pl.pallas_call(
        _fused_body,
        grid=(s // ts,),
        in_specs=[
            pl.BlockSpec((ts, b, d), lambda i: (i, 0, 0)),
            pl.BlockSpec((e, d), lambda i: (0, 0)),
        ],
        out_specs=[
            pl.BlockSpec((ts, b, e), lambda i: (i, 0, 0)),
            pl.BlockSpec((ts, b, e), lambda i: (i, 0, 0)),
        ],
        out_shape=[
            jax.ShapeDtypeStruct((s, b, e), jnp.float32),
            jax.ShapeDtypeStruct((s, b, e), jnp.bool_),
        ],
        compiler_params=pltpu.CompilerParams(
            dimension_semantics=("arbitrary",),
        ),
    )(h, w)
    return probs3.reshape(s * b, e), map3.reshape(s * b, e)


def kernel(hidden_states, router_weight):
    return _route_fused3d(
        hidden_states.astype(jnp.float32),
        router_weight.astype(jnp.float32),
        ts=512,
    )


# speedup vs baseline: 4.0053x; 1.2335x over previous
import functools

import jax
import jax.numpy as jnp
from jax import lax
from jax.experimental import pallas as pl
from jax.experimental.pallas import tpu as pltpu
from jax.experimental.pallas import tpu_sc as plsc

_NUM_CORES = 2
_NUM_SUBCORES = 16
_NUM_WORKERS = _NUM_CORES * _NUM_SUBCORES
_LANES = 16


def _top2(logits, e):
    tt = logits.shape[0]
    iota = jax.lax.broadcasted_iota(jnp.int32, (tt, e), 1)
    m1 = jnp.max(logits, axis=1, keepdims=True)
    idx1 = jnp.min(jnp.where(logits == m1, iota, e), axis=1, keepdims=True)
    masked = jnp.where(iota == idx1, -jnp.inf, logits)
    m2 = jnp.max(masked, axis=1, keepdims=True)
    idx2 = jnp.min(jnp.where(masked == m2, iota, e), axis=1, keepdims=True)
    return iota, m1, idx1, m2, idx2


def _fused_body(x_ref, w_ref, probs_ref, map_ref):
    ts, b, d = x_ref.shape
    e = w_ref.shape[0]
    x = x_ref[...].reshape(ts * b, d)
    logits = jax.lax.dot_general(
        x, w_ref[...], (((1,), (1,)), ((), ())),
        preferred_element_type=jnp.float32,
    )
    iota, m1, idx1, m2, idx2 = _top2(logits, e)
    t = jnp.exp(m2 - m1)
    denom = 1.0 + t
    p1 = 1.0 / denom
    p2 = t / denom
    probs = jnp.where(iota == idx1, p1, jnp.where(iota == idx2, p2, 0.0))
    rmap = (iota == idx1) | (iota == idx2)
    probs_ref[...] = probs
    map_ref[...] = rmap


@functools.partial(jax.jit, static_argnames=("ts",))
def _route_fused3d(h, w, ts):
    s, b, d = h.shape
    e = w.shape[0]
    return pl.pallas_call(
        _fused_body,
        grid=(s // ts,),
        in_specs=[
            pl.BlockSpec((ts, b, d), lambda i: (i, 0, 0)),
            pl.BlockSpec((e, d), lambda i: (0, 0)),
        ],
        out_specs=[
            pl.BlockSpec((ts * b, e), lambda i: (i, 0)),
            pl.BlockSpec((ts * b, e), lambda i: (i, 0)),
        ],
        out_shape=[
            jax.ShapeDtypeStruct((s * b, e), jnp.float32),
            jax.ShapeDtypeStruct((s * b, e), jnp.bool_),
        ],
        compiler_params=pltpu.CompilerParams(
            dimension_semantics=("arbitrary",),
        ),
    )(h, w)


def kernel(hidden_states, router_weight):
    return _route_fused3d(
        hidden_states.astype(jnp.float32),
        router_weight.astype(jnp.float32),
        ts=512,
    )
